# Initial kernel scaffold; baseline (speedup 1.0000x reference)
import jax
import jax.numpy as jnp
from jax.experimental import pallas as pl


def kernel(input_ids, tok_table, pos_table):
    raise NotImplementedError("write your pallas kernel here")


# speedup vs baseline: 1.9363x; 1.9363x over previous
import functools

import jax
import jax.numpy as jnp
from jax import lax
from jax.experimental import pallas as pl
from jax.experimental.pallas import tpu as pltpu
from jax.experimental.pallas import tpu_sc as plsc

D = 128
T = 200
LANES = 16
CHUNK = 128


@functools.lru_cache(maxsize=None)
def _build(n_rows: int):
    info = plsc.get_sparse_core_info()
    nw = info.num_cores * info.num_subcores
    assert n_rows % (nw * CHUNK) == 0
    rows_per_w = n_rows // nw
    n_chunks = rows_per_w // CHUNK
    mesh = plsc.VectorSubcoreMesh(core_axis_name="c", subcore_axis_name="s")

    @functools.partial(
        pl.kernel,
        mesh=mesh,
        out_type=jax.ShapeDtypeStruct((n_rows, D), jnp.float32),
        scratch_types=[
            pltpu.VMEM((CHUNK,), jnp.int32),
            pltpu.VMEM((CHUNK, D), jnp.float32),
            pltpu.VMEM((2 * T, D), jnp.float32),
            pltpu.SemaphoreType.DMA,
        ],
    )
    def k(ids_hbm, tok_hbm, pos_hbm, out_hbm, idx_v, rows_v, pos_v, sem):
        wid = lax.axis_index("s") * info.num_cores + lax.axis_index("c")
        base = wid * rows_per_w
        pltpu.sync_copy(pos_hbm, pos_v.at[pl.ds(0, T)])
        pltpu.sync_copy(pos_hbm, pos_v.at[pl.ds(T, T)])

        def chunk_body(c, _):
            rbase = base + c * CHUNK
            pltpu.sync_copy(ids_hbm.at[pl.ds(rbase, CHUNK)], idx_v)
            pltpu.async_copy(tok_hbm.at[idx_v], rows_v, sem).wait()
            po = lax.rem(rbase, T)

            def row_body(r, _):
                src = po + r
                for j in range(D // LANES):
                    sl = pl.ds(j * LANES, LANES)
                    rows_v[r, sl] = rows_v[r, sl] + pos_v[src, sl]
                return 0

            lax.fori_loop(0, CHUNK, row_body, 0, unroll=2)
            pltpu.sync_copy(rows_v, out_hbm.at[pl.ds(rbase, CHUNK)])
            return 0

        lax.fori_loop(0, n_chunks, chunk_body, 0)

    return k


def kernel(input_ids, tok_table, pos_table):
    b, t = input_ids.shape
    ids = input_ids.reshape(-1).astype(jnp.int32)
    out = _build(b * t)(ids, tok_table, pos_table)
    return out.reshape(b, t, D)


# speedup vs baseline: 2.2480x; 1.1610x over previous
import functools

import jax
import jax.numpy as jnp
from jax import lax
from jax.experimental import pallas as pl
from jax.experimental.pallas import tpu as pltpu
from jax.experimental.pallas import tpu_sc as plsc

D = 128
T = 200
LANES = 16
CHUNK = 64
NBUF = 4


@functools.lru_cache(maxsize=None)
def _build(n_rows: int):
    info = plsc.get_sparse_core_info()
    nw = info.num_cores * info.num_subcores
    rows_per_w = n_rows // nw
    n_chunks = rows_per_w // CHUNK
    assert n_rows == nw * n_chunks * CHUNK
    assert (n_chunks - 4) % NBUF == 0
    mesh = plsc.VectorSubcoreMesh(core_axis_name="c", subcore_axis_name="s")

    @functools.partial(
        pl.kernel,
        mesh=mesh,
        out_type=jax.ShapeDtypeStruct((n_rows, D), jnp.float32),
        scratch_types=[
            pltpu.VMEM((NBUF, CHUNK), jnp.int32),
            *([pltpu.VMEM((CHUNK, D), jnp.float32)] * NBUF),
            pltpu.VMEM((2 * T, D), jnp.float32),
            *([pltpu.SemaphoreType.DMA] * (2 * NBUF)),
        ],
    )
    def k(ids_hbm, tok_hbm, pos_hbm, out_hbm, idx_v, r0, r1, r2, r3,
          pos_v, g0, g1, g2, g3, s0, s1, s2, s3):
        rows = (r0, r1, r2, r3)
        gsem = (g0, g1, g2, g3)
        ssem = (s0, s1, s2, s3)
        wid = lax.axis_index("s") * info.num_cores + lax.axis_index("c")
        base = wid * rows_per_w

        def start_gather(c, b):
            pltpu.sync_copy(ids_hbm.at[pl.ds(base + c * CHUNK, CHUNK)],
                            idx_v.at[b])
            pltpu.async_copy(tok_hbm.at[idx_v.at[b]], rows[b], gsem[b])

        def wait_gather(b):
            pltpu.make_async_copy(tok_hbm.at[idx_v.at[b]], rows[b],
                                  gsem[b]).wait()

        def start_scatter(c, b):
            pltpu.async_copy(rows[b], out_hbm.at[pl.ds(base + c * CHUNK,
                                                       CHUNK)], ssem[b])

        def wait_scatter(b):
            pltpu.make_async_copy(rows[b], out_hbm.at[pl.ds(0, CHUNK)],
                                  ssem[b]).wait()

        def add_pos(c, b):
            po = lax.rem(base + c * CHUNK, T)

            def row_body(r, _):
                src = po + r
                for j in range(D // LANES):
                    sl = pl.ds(j * LANES, LANES)
                    rows[b][r, sl] = rows[b][r, sl] + pos_v[src, sl]
                return 0

            lax.fori_loop(0, CHUNK, row_body, 0, unroll=2)

        pltpu.sync_copy(pos_hbm, pos_v.at[pl.ds(0, T)])
        pltpu.sync_copy(pos_hbm, pos_v.at[pl.ds(T, T)])

        for j in range(NBUF - 1):
            start_gather(j, j)

        wait_gather(0)
        add_pos(0, 0)
        start_scatter(0, 0)
        start_gather(NBUF - 1, NBUF - 1)

        def group(i, _):
            c0 = 1 + i * NBUF
            for j in range(NBUF):
                b = (1 + j) % NBUF
                c = c0 + j
                wait_gather(b)
                add_pos(c, b)
                start_scatter(c, b)
                wait_scatter((b + NBUF - 1) % NBUF)
                start_gather(c + NBUF - 1, (b + NBUF - 1) % NBUF)
            return 0

        lax.fori_loop(0, (n_chunks - NBUF) // NBUF, group, 0)

        for j in range(NBUF - 1):
            c = n_chunks - (NBUF - 1) + j
            b = c % NBUF
            wait_gather(b)
            add_pos(c, b)
            start_scatter(c, b)

        for b in range(NBUF):
            wait_scatter(b)

    return k


def kernel(input_ids, tok_table, pos_table):
    b, t = input_ids.shape
    ids = input_ids.reshape(-1).astype(jnp.int32)
    out = _build(b * t)(ids, tok_table, pos_table)
    return out.reshape(b, t, D)


# speedup vs baseline: 6.3519x; 2.8255x over previous
import functools

import jax
import jax.numpy as jnp
from jax import lax
from jax.experimental import pallas as pl
from jax.experimental.pallas import tpu as pltpu
from jax.experimental.pallas import tpu_sc as plsc

D = 128
T = 200
LANES = 16
CHUNK = 64
NBUF = 4


@functools.lru_cache(maxsize=None)
def _build(n_rows: int):
    info = plsc.get_sparse_core_info()
    nw = info.num_cores * info.num_subcores
    rows_per_w = n_rows // nw
    n_chunks = rows_per_w // CHUNK
    assert n_rows == nw * n_chunks * CHUNK
    assert (n_chunks - 4) % NBUF == 0
    mesh = plsc.VectorSubcoreMesh(core_axis_name="c", subcore_axis_name="s")

    @functools.partial(
        pl.kernel,
        mesh=mesh,
        out_type=jax.ShapeDtypeStruct((n_rows, D), jnp.float32),
        scratch_types=[
            pltpu.VMEM((NBUF, CHUNK), jnp.int32),
            *([pltpu.VMEM((CHUNK, D), jnp.float32)] * NBUF),
            pltpu.VMEM_SHARED((2 * T, D), jnp.float32),
            *([pltpu.SemaphoreType.DMA] * (2 * NBUF)),
        ],
    )
    def k(ids_hbm, tok_hbm, pos_hbm, out_hbm, idx_v, r0, r1, r2, r3,
          pos_v, g0, g1, g2, g3, s0, s1, s2, s3):
        rows = (r0, r1, r2, r3)
        gsem = (g0, g1, g2, g3)
        ssem = (s0, s1, s2, s3)
        wid = lax.axis_index("s") * info.num_cores + lax.axis_index("c")
        base = wid * rows_per_w

        def start_gather(c, b):
            po = lax.rem(base + c * CHUNK, T)
            pltpu.sync_copy(pos_v.at[pl.ds(po, CHUNK)], rows[b])
            pltpu.sync_copy(ids_hbm.at[pl.ds(base + c * CHUNK, CHUNK)],
                            idx_v.at[b])
            pltpu.async_copy(tok_hbm.at[idx_v.at[b]], rows[b], gsem[b],
                             add=True)

        def wait_gather(b):
            pltpu.make_async_copy(tok_hbm.at[idx_v.at[b]], rows[b],
                                  gsem[b]).wait()

        def start_scatter(c, b):
            pltpu.async_copy(rows[b], out_hbm.at[pl.ds(base + c * CHUNK,
                                                       CHUNK)], ssem[b])

        def wait_scatter(b):
            pltpu.make_async_copy(rows[b], out_hbm.at[pl.ds(0, CHUNK)],
                                  ssem[b]).wait()

        @pl.when(lax.axis_index("s") == 0)
        def _stage_pos():
            pltpu.sync_copy(pos_hbm, pos_v.at[pl.ds(0, T)])
            pltpu.sync_copy(pos_hbm, pos_v.at[pl.ds(T, T)])

        plsc.subcore_barrier()

        for j in range(NBUF - 1):
            start_gather(j, j)

        wait_gather(0)
        start_scatter(0, 0)
        start_gather(NBUF - 1, NBUF - 1)

        def group(i, _):
            c0 = 1 + i * NBUF
            for j in range(NBUF):
                b = (1 + j) % NBUF
                c = c0 + j
                wait_gather(b)
                start_scatter(c, b)
                wait_scatter((b + NBUF - 1) % NBUF)
                start_gather(c + NBUF - 1, (b + NBUF - 1) % NBUF)
            return 0

        lax.fori_loop(0, (n_chunks - NBUF) // NBUF, group, 0)

        for j in range(NBUF - 1):
            c = n_chunks - (NBUF - 1) + j
            b = c % NBUF
            wait_gather(b)
            start_scatter(c, b)

        for b in range(NBUF):
            wait_scatter(b)

    return k


def kernel(input_ids, tok_table, pos_table):
    b, t = input_ids.shape
    ids = input_ids.reshape(-1).astype(jnp.int32)
    out = _build(b * t)(ids, tok_table, pos_table)
    return out.reshape(b, t, D)


# speedup vs baseline: 7.1420x; 1.1244x over previous
import functools

import jax
import jax.numpy as jnp
from jax import lax
from jax.experimental import pallas as pl
from jax.experimental.pallas import tpu as pltpu
from jax.experimental.pallas import tpu_sc as plsc

D = 128
T = 200
LANES = 16
CHUNK = 128
NBUF = 5


@functools.lru_cache(maxsize=None)
def _build(n_rows: int):
    info = plsc.get_sparse_core_info()
    nw = info.num_cores * info.num_subcores
    rows_per_w = n_rows // nw
    n_chunks = rows_per_w // CHUNK
    assert n_rows == nw * n_chunks * CHUNK
    assert (n_chunks - NBUF) % NBUF == 0
    mesh = plsc.VectorSubcoreMesh(core_axis_name="c", subcore_axis_name="s")

    @functools.partial(
        pl.kernel,
        mesh=mesh,
        out_type=jax.ShapeDtypeStruct((n_rows, D), jnp.float32),
        scratch_types=[
            pltpu.VMEM((NBUF, CHUNK), jnp.int32),
            *([pltpu.VMEM((CHUNK, D), jnp.float32)] * NBUF),
            pltpu.VMEM_SHARED((2 * T, D), jnp.float32),
            *([pltpu.SemaphoreType.DMA] * (2 * NBUF)),
        ],
    )
    def k(ids_hbm, tok_hbm, pos_hbm, out_hbm, idx_v, *bufs):
        rows = bufs[:NBUF]
        pos_v = bufs[NBUF]
        gsem = bufs[NBUF + 1:2 * NBUF + 1]
        ssem = bufs[2 * NBUF + 1:]
        wid = lax.axis_index("s") * info.num_cores + lax.axis_index("c")
        base = wid * rows_per_w

        def start_gather(c, b):
            po = lax.rem(base + c * CHUNK, T)
            pltpu.sync_copy(pos_v.at[pl.ds(po, CHUNK)], rows[b])
            pltpu.sync_copy(ids_hbm.at[pl.ds(base + c * CHUNK, CHUNK)],
                            idx_v.at[b])
            pltpu.async_copy(tok_hbm.at[idx_v.at[b]], rows[b], gsem[b],
                             add=True)

        def wait_gather(b):
            pltpu.make_async_copy(tok_hbm.at[idx_v.at[b]], rows[b],
                                  gsem[b]).wait()

        def start_scatter(c, b):
            pltpu.async_copy(rows[b], out_hbm.at[pl.ds(base + c * CHUNK,
                                                       CHUNK)], ssem[b])

        def wait_scatter(b):
            pltpu.make_async_copy(rows[b], out_hbm.at[pl.ds(0, CHUNK)],
                                  ssem[b]).wait()

        @pl.when(lax.axis_index("s") == 0)
        def _stage_pos():
            pltpu.sync_copy(pos_hbm, pos_v.at[pl.ds(0, T)])
            pltpu.sync_copy(pos_hbm, pos_v.at[pl.ds(T, T)])

        plsc.subcore_barrier()

        for j in range(NBUF - 1):
            start_gather(j, j)

        wait_gather(0)
        start_scatter(0, 0)
        start_gather(NBUF - 1, NBUF - 1)

        def group(i, _):
            c0 = 1 + i * NBUF
            for j in range(NBUF):
                b = (1 + j) % NBUF
                c = c0 + j
                wait_gather(b)
                start_scatter(c, b)
                wait_scatter((b + NBUF - 1) % NBUF)
                start_gather(c + NBUF - 1, (b + NBUF - 1) % NBUF)
            return 0

        lax.fori_loop(0, (n_chunks - NBUF) // NBUF, group, 0)

        for j in range(NBUF - 1):
            c = n_chunks - (NBUF - 1) + j
            b = c % NBUF
            wait_gather(b)
            start_scatter(c, b)

        for b in range(NBUF):
            wait_scatter(b)

    return k


def kernel(input_ids, tok_table, pos_table):
    b, t = input_ids.shape
    ids = input_ids.reshape(-1).astype(jnp.int32)
    out = _build(b * t)(ids, tok_table, pos_table)
    return out.reshape(b, t, D)
